# Initial kernel scaffold; baseline (speedup 1.0000x reference)
import jax
import jax.numpy as jnp
from jax.experimental import pallas as pl


def kernel(C):
    raise NotImplementedError("write your pallas kernel here")


# speedup vs baseline: 2.1077x; 2.1077x over previous
import functools

import jax
import jax.numpy as jnp
import numpy as np
from jax import lax
from jax.experimental import pallas as pl
from jax.experimental.pallas import tpu as pltpu
from jax.experimental.pallas import tpu_sc as plsc

_A_IDX = [0, 0, 0, 0, 0, 0, 1, 1, 1, 1, 1, 2, 2, 2, 2, 1, 1, 1, 0, 0, 0]
_B_IDX = [0, 0, 0, 0, 0, 0, 1, 1, 1, 1, 1, 2, 2, 2, 2, 2, 2, 2, 2, 2, 1]
_C_IDX = [0, 1, 2, 1, 0, 0, 1, 2, 1, 0, 0, 2, 1, 0, 0, 1, 0, 0, 0, 0, 0]
_D_IDX = [0, 1, 2, 2, 2, 1, 1, 2, 2, 2, 1, 2, 2, 2, 1, 2, 2, 1, 2, 1, 1]


def _tables():
    flat = [27 * a + 9 * b + 3 * c + d
            for a, b, c, d in zip(_A_IDX, _B_IDX, _C_IDX, _D_IDX)]
    rows, cols = np.triu_indices(6)
    s2 = np.sqrt(2)
    m = np.array([[1, 1, 1, s2, s2, s2],
                  [1, 1, 1, s2, s2, s2],
                  [1, 1, 1, s2, s2, s2],
                  [s2, s2, s2, 2, 2, 2],
                  [s2, s2, s2, 2, 2, 2],
                  [s2, s2, s2, 2, 2, 2]], dtype=np.float32)
    comp_of = {}
    for k, (r, c) in enumerate(zip(rows, cols)):
        comp_of[(r, c)] = k
        comp_of[(c, r)] = k
    out_comp = [comp_of[(i, j)] for i in range(6) for j in range(6)]
    out_scale = [float(m[i, j]) for i in range(6) for j in range(6)]
    return flat, out_comp, out_scale

_FLAT, _OUT_COMP, _OUT_SCALE = _tables()

_B = 500000
_S = 400
_NCHUNK = _B // _S
_NW = 32
_ITERS = -(-_NCHUNK // _NW)


def _body(c_hbm, out_hbm, in_v, out_v):
    wid = lax.axis_index("s") * 2 + lax.axis_index("c")
    lane = lax.iota(jnp.int32, 16)

    def chunk_step(i, _):
        chunk = wid + i * _NW

        @pl.when(chunk < _NCHUNK)
        def _():
            base = chunk * _S
            pltpu.sync_copy(c_hbm.at[pl.ds(base, _S)], in_v)

            def group_step(g, _):
                sidx = lane + g * 16
                vals = [plsc.load_gather(
                            in_v, [sidx, jnp.full((16,), _FLAT[k], jnp.int32)])
                        for k in range(21)]
                for j in range(36):
                    plsc.store_scatter(
                        out_v, [sidx, jnp.full((16,), j, jnp.int32)],
                        vals[_OUT_COMP[j]] * _OUT_SCALE[j])
                return 0

            lax.fori_loop(0, _S // 16, group_step, 0)
            pltpu.sync_copy(out_v, out_hbm.at[pl.ds(base, _S)])

        return 0

    lax.fori_loop(0, _ITERS, chunk_step, 0)


@jax.jit
def kernel(C):
    c2 = C.reshape(_B, 81)
    mesh = plsc.VectorSubcoreMesh(core_axis_name="c", subcore_axis_name="s")
    out = pl.kernel(
        _body,
        out_type=jax.ShapeDtypeStruct((_B, 36), jnp.float32),
        mesh=mesh,
        scratch_types=[
            pltpu.VMEM((_S, 81), jnp.float32),
            pltpu.VMEM((_S, 36), jnp.float32),
        ],
        compiler_params=pltpu.CompilerParams(needs_layout_passes=False),
    )(c2)
    return out.reshape(_B, 6, 6)
